# baseline (speedup 1.0000x reference)
import functools

import jax
import jax.numpy as jnp
from jax import lax
from jax.experimental import pallas as pl
from jax.experimental.pallas import tpu as pltpu
from jax.experimental.pallas import tpu_sc as plsc

_C = 16


@functools.partial(jax.jit, static_argnums=(2, 3))
def _sc_gather(table, idx, NC, NW):
    B = idx.shape[0]
    D = table.shape[1]
    bpw = B // NW
    mesh = plsc.VectorSubcoreMesh(core_axis_name="c", subcore_axis_name="s")

    @functools.partial(
        pl.kernel,
        mesh=mesh,
        out_type=jax.ShapeDtypeStruct((B, D), jnp.float32),
        scratch_types=[
            pltpu.VMEM((bpw,), jnp.int32),
            pltpu.VMEM((bpw, D), jnp.float32),
            pltpu.SemaphoreType.DMA,
        ],
    )
    def gather_kernel(table_hbm, idx_hbm, out_hbm, idx_v, rows_v, sem):
        wid = lax.axis_index("s") * NC + lax.axis_index("c")
        base = wid * bpw
        with jax.named_scope("idx_load"):
            pltpu.sync_copy(idx_hbm.at[pl.ds(base, bpw)], idx_v)

        def fire(g, carry):
            off = g * _C
            v16 = idx_v[pl.ds(off, _C)]
            for n in range(_C):
                pltpu.async_copy(table_hbm.at[pl.ds(v16[n], 1)],
                                 rows_v.at[pl.ds(off + n, 1)], sem)
            return carry

        with jax.named_scope("fire"):
            lax.fori_loop(0, bpw // _C, fire, 0)
        with jax.named_scope("drain"):
            pltpu.make_async_copy(
                out_hbm.at[pl.ds(base, bpw)], rows_v, sem).wait()
        with jax.named_scope("writeback"):
            pltpu.sync_copy(rows_v, out_hbm.at[pl.ds(base, bpw)])

    return gather_kernel(table, idx)


def kernel(batch, table):
    info = plsc.get_sparse_core_info()
    NW = info.num_cores * info.num_subcores
    idx = batch[0].astype(jnp.int32)
    return _sc_gather(table, idx, info.num_cores, NW)


# speedup vs baseline: 1.0028x; 1.0028x over previous
import functools

import jax
import jax.numpy as jnp
from jax import lax
from jax.experimental import pallas as pl
from jax.experimental.pallas import tpu as pltpu
from jax.experimental.pallas import tpu_sc as plsc

_C = 16


@functools.partial(jax.jit, static_argnums=(2, 3))
def _sc_gather(table, idx, NC, NW):
    B = idx.shape[0]
    D = table.shape[1]
    bpw = B // NW
    mesh = plsc.VectorSubcoreMesh(core_axis_name="c", subcore_axis_name="s")

    @functools.partial(
        pl.kernel,
        mesh=mesh,
        out_type=jax.ShapeDtypeStruct((B, D), jnp.float32),
        scratch_types=[
            pltpu.VMEM((bpw,), jnp.int32),
            pltpu.VMEM((bpw, D), jnp.float32),
            pltpu.SemaphoreType.DMA,
        ],
    )
    def gather_kernel(table_hbm, idx_hbm, out_hbm, idx_v, rows_v, sem):
        wid = lax.axis_index("s") * NC + lax.axis_index("c")
        base = wid * bpw
        with jax.named_scope("idx_load"):
            pltpu.sync_copy(idx_hbm.at[pl.ds(base, bpw)], idx_v)

        def fire(g, carry):
            off = g * _C
            v16 = idx_v[pl.ds(off, _C)]
            for n in range(_C):
                pltpu.async_copy(table_hbm.at[pl.ds(v16[n], 1)],
                                 rows_v.at[pl.ds(off + n, 1)], sem)
            return carry

        with jax.named_scope("fire"):
            lax.fori_loop(0, bpw // _C, fire, 0)

        def drain(g, carry):
            off = g * _C
            for n in range(_C):
                pltpu.make_async_copy(
                    table_hbm.at[pl.ds(0, 1)],
                    rows_v.at[pl.ds(off + n, 1)], sem).wait()
            return carry

        with jax.named_scope("drain"):
            lax.fori_loop(0, bpw // _C, drain, 0)
        with jax.named_scope("writeback"):
            pltpu.sync_copy(rows_v, out_hbm.at[pl.ds(base, bpw)])

    return gather_kernel(table, idx)


def kernel(batch, table):
    info = plsc.get_sparse_core_info()
    NW = info.num_cores * info.num_subcores
    idx = batch[0].astype(jnp.int32)
    return _sc_gather(table, idx, info.num_cores, NW)
